# Initial kernel scaffold; baseline (speedup 1.0000x reference)
import jax
import jax.numpy as jnp
from jax.experimental import pallas as pl


def kernel(x, prob):
    raise NotImplementedError("write your pallas kernel here")


# speedup vs baseline: 1.0192x; 1.0192x over previous
import functools

import jax
import jax.numpy as jnp
from jax import lax
from jax.experimental import pallas as pl
from jax.experimental.pallas import tpu as pltpu
from jax.experimental.pallas import tpu_sc as plsc

_D = 1000
_N = 1024 * 50
_NC, _NS = 2, 16
_NW = _NC * _NS
_RPW = _N // _NW
_CHUNK = 80
_NCHUNK = _RPW // _CHUNK


def _sc_gather(x_flat, prob):
  mesh = plsc.VectorSubcoreMesh(core_axis_name="c", subcore_axis_name="s")

  @functools.partial(
      pl.kernel,
      out_type=jax.ShapeDtypeStruct((_N, _D), jnp.float32),
      mesh=mesh,
      scratch_types=[
          pltpu.VMEM((_RPW,), jnp.int32),
          pltpu.VMEM((_CHUNK, _D), jnp.float32),
          pltpu.SemaphoreType.DMA,
      ],
      compiler_params=pltpu.CompilerParams(use_tc_tiling_on_sc=False),
  )
  def body(idx_hbm, table_hbm, out_hbm, idx_v, rows, gsem):
    wid = lax.axis_index("s") * _NC + lax.axis_index("c")
    base = wid * _RPW
    pltpu.sync_copy(idx_hbm.at[pl.ds(base, _RPW)], idx_v)

    def step(c, carry):
      off = pl.multiple_of(c * _CHUNK, _CHUNK)
      pltpu.async_copy(
          table_hbm.at[idx_v.at[pl.ds(off, _CHUNK)]], rows, gsem).wait()
      pltpu.sync_copy(rows, out_hbm.at[pl.ds(base + off, _CHUNK)])
      return carry

    lax.fori_loop(0, _NCHUNK, step, 0)

  return body(x_flat, prob)


def kernel(x, prob):
  x_flat = x.reshape(-1)
  out = _sc_gather(x_flat, prob)
  return out.reshape(x.shape[0], x.shape[1], _D)


# speedup vs baseline: 1.0221x; 1.0029x over previous
import functools

import jax
import jax.numpy as jnp
from jax import lax
from jax.experimental import pallas as pl
from jax.experimental.pallas import tpu as pltpu
from jax.experimental.pallas import tpu_sc as plsc

_D = 1000
_N = 1024 * 50
_NC, _NS = 2, 16
_NW = _NC * _NS
_RPW = _N // _NW
_CHUNK = 40
_NCHUNK = _RPW // _CHUNK


def _sc_gather(x_flat, prob):
  mesh = plsc.VectorSubcoreMesh(core_axis_name="c", subcore_axis_name="s")

  @functools.partial(
      pl.kernel,
      out_type=jax.ShapeDtypeStruct((_N, _D), jnp.float32),
      mesh=mesh,
      scratch_types=[
          pltpu.VMEM((_RPW,), jnp.int32),
          pltpu.VMEM((_CHUNK, _D), jnp.float32),
          pltpu.VMEM((_CHUNK, _D), jnp.float32),
          pltpu.SemaphoreType.DMA,
          pltpu.SemaphoreType.DMA,
          pltpu.SemaphoreType.DMA,
          pltpu.SemaphoreType.DMA,
      ],
      compiler_params=pltpu.CompilerParams(use_tc_tiling_on_sc=False),
  )
  def body(idx_hbm, table_hbm, out_hbm, idx_v, rows0, rows1, g0, g1, s0, s1):
    wid = lax.axis_index("s") * _NC + lax.axis_index("c")
    base = wid * _RPW
    pltpu.sync_copy(idx_hbm.at[pl.ds(base, _RPW)], idx_v)

    bufs = (rows0, rows1)
    gsems = (g0, g1)
    ssems = (s0, s1)

    def gather(c, p):
      off = pl.multiple_of(c * _CHUNK, _CHUNK)
      return pltpu.make_async_copy(
          table_hbm.at[idx_v.at[pl.ds(off, _CHUNK)]], bufs[p], gsems[p])

    def scatter(c, p):
      off = pl.multiple_of(base + c * _CHUNK, _CHUNK)
      return pltpu.make_async_copy(
          bufs[p], out_hbm.at[pl.ds(off, _CHUNK)], ssems[p])

    gather(0, 0).start()
    gather(1, 1).start()

    def step(jj, carry):
      c0 = 2 * jj
      gather(c0, 0).wait()
      scatter(c0, 0).start()
      gather(c0 + 1, 1).wait()
      scatter(c0 + 1, 1).start()
      scatter(c0, 0).wait()
      gather(c0 + 2, 0).start()
      scatter(c0 + 1, 1).wait()
      gather(c0 + 3, 1).start()
      return carry

    lax.fori_loop(0, _NCHUNK // 2 - 1, step, 0)

    cl = _NCHUNK - 2
    gather(cl, 0).wait()
    scatter(cl, 0).start()
    gather(cl + 1, 1).wait()
    scatter(cl + 1, 1).start()
    scatter(cl, 0).wait()
    scatter(cl + 1, 1).wait()

  return body(x_flat, prob)


def kernel(x, prob):
  x_flat = x.reshape(-1)
  out = _sc_gather(x_flat, prob)
  return out.reshape(x.shape[0], x.shape[1], _D)
